# baseline (speedup 1.0000x reference)
import functools

import jax
import jax.numpy as jnp
from jax import lax
from jax.experimental import pallas as pl
from jax.experimental.pallas import tpu as pltpu
from jax.experimental.pallas import tpu_sc as plsc

_B, _N, _D_IN, _D_LAT, _K, _NUM_CB = 16, 576, 768, 256, 8192, 4
_BETA = 0.25
_M = _B * _N


def _proj_in_body(z_ref, w_ref, b_ref, o_ref):
    o_ref[...] = (
        jnp.dot(z_ref[...], w_ref[...], preferred_element_type=jnp.float32)
        + b_ref[...]
    )


def _proj_in(z2, w, b):
    tm = 1024
    return pl.pallas_call(
        _proj_in_body,
        grid=(_M // tm,),
        in_specs=[
            pl.BlockSpec((tm, _D_IN), lambda i: (i, 0)),
            pl.BlockSpec((_D_IN, _D_LAT), lambda i: (0, 0)),
            pl.BlockSpec((1, _D_LAT), lambda i: (0, 0)),
        ],
        out_specs=pl.BlockSpec((tm, _D_LAT), lambda i: (i, 0)),
        out_shape=jax.ShapeDtypeStruct((_M, _D_LAT), jnp.float32),
    )(z2, w, b)


def _e2_body(cb_ref, e2_ref):
    e = cb_ref[0]
    e2_ref[0, 0, :] = jnp.sum(e * e, axis=1)


def _e2_all(codebooks):
    out = pl.pallas_call(
        _e2_body,
        grid=(_NUM_CB,),
        in_specs=[pl.BlockSpec((1, _K, _D_LAT), lambda h: (h, 0, 0))],
        out_specs=pl.BlockSpec((1, 1, _K), lambda h: (h, 0, 0)),
        out_shape=jax.ShapeDtypeStruct((_NUM_CB, 1, _K), jnp.float32),
    )(codebooks)
    return [out[h] for h in range(_NUM_CB)]


_W = 128
_C = _K // _W


def _dist_core(rn, e_ref, e2_ref, idx_ref):
    a = jnp.sum(rn * rn, axis=1, keepdims=True)
    rs = rn * (-2.0)
    s2 = lax.dot_general(
        rs, e_ref[...], (((1,), (1,)), ((), ())),
        preferred_element_type=jnp.float32,
    )
    d = (a + e2_ref[...]) + s2
    big = float(_K)
    colmin = d[:, 0:_W]
    for c in range(1, _C):
        colmin = jnp.minimum(colmin, d[:, c * _W:(c + 1) * _W])
    runc = jnp.where(d[:, 0:_W] == colmin, 0.0, big)
    for c in range(1, _C):
        cand = jnp.where(d[:, c * _W:(c + 1) * _W] == colmin, float(c), big)
        runc = jnp.minimum(runc, cand)
    rowmin = jnp.min(colmin, axis=1, keepdims=True)
    lane = lax.broadcasted_iota(jnp.int32, (1, _W), 1).astype(jnp.float32)
    key = jnp.where(colmin == rowmin, runc * float(_W) + lane, big * big)
    idxf = jnp.min(key, axis=1)
    idx_ref[...] = idxf.astype(jnp.int32)


def _upd_core(r, g, act):
    delta = g - r
    zql = jnp.where(act, r + delta, jnp.zeros_like(r))
    m = jnp.mean(delta * delta, axis=1, keepdims=True)
    lossl = jnp.where(act, m + m * _BETA, jnp.zeros_like(m))
    return zql, lossl, r - zql


_TM = 1024


def _proj_dist0_body(z_ref, w_ref, b_ref, e_ref, e2_ref, r_ref, idx_ref):
    r0 = (
        jnp.dot(z_ref[...], w_ref[...], preferred_element_type=jnp.float32)
        + b_ref[...]
    )
    r_ref[...] = r0
    _dist_core(r0, e_ref, e2_ref, idx_ref)


def _proj_dist0(z2, w, b, e, e2):
    return pl.pallas_call(
        _proj_dist0_body,
        grid=(_M // _TM,),
        in_specs=[
            pl.BlockSpec((_TM, _D_IN), lambda i: (i, 0)),
            pl.BlockSpec((_D_IN, _D_LAT), lambda i: (0, 0)),
            pl.BlockSpec((1, _D_LAT), lambda i: (0, 0)),
            pl.BlockSpec((_K, _D_LAT), lambda i: (0, 0)),
            pl.BlockSpec((1, _K), lambda i: (0, 0)),
        ],
        out_specs=[
            pl.BlockSpec((_TM, _D_LAT), lambda i: (i, 0)),
            pl.BlockSpec((_TM,), lambda i: (i,)),
        ],
        out_shape=[
            jax.ShapeDtypeStruct((_M, _D_LAT), jnp.float32),
            jax.ShapeDtypeStruct((_M,), jnp.int32),
        ],
        compiler_params=pltpu.CompilerParams(
            dimension_semantics=("parallel",)),
    )(z2, w, b, e, e2)


def _upd_dist_body(r_ref, g_ref, act_ref, e_ref, e2_ref,
                   zql_ref, loss_ref, rn_ref, idx_ref):
    act = act_ref[0] != 0
    zql, lossl, rn = _upd_core(r_ref[...], g_ref[...], act)
    zql_ref[...] = zql
    loss_ref[...] = lossl
    rn_ref[...] = rn
    _dist_core(rn, e_ref, e2_ref, idx_ref)


def _upd_dist(r, g, act, e, e2):
    return pl.pallas_call(
        _upd_dist_body,
        grid=(_M // _TM,),
        in_specs=[
            pl.BlockSpec((_TM, _D_LAT), lambda i: (i, 0)),
            pl.BlockSpec((_TM, _D_LAT), lambda i: (i, 0)),
            pl.BlockSpec(memory_space=pltpu.SMEM),
            pl.BlockSpec((_K, _D_LAT), lambda i: (0, 0)),
            pl.BlockSpec((1, _K), lambda i: (0, 0)),
        ],
        out_specs=[
            pl.BlockSpec((_TM, _D_LAT), lambda i: (i, 0)),
            pl.BlockSpec((_TM, 1), lambda i: (i, 0)),
            pl.BlockSpec((_TM, _D_LAT), lambda i: (i, 0)),
            pl.BlockSpec((_TM,), lambda i: (i,)),
        ],
        out_shape=[
            jax.ShapeDtypeStruct((_M, _D_LAT), jnp.float32),
            jax.ShapeDtypeStruct((_M, 1), jnp.float32),
            jax.ShapeDtypeStruct((_M, _D_LAT), jnp.float32),
            jax.ShapeDtypeStruct((_M,), jnp.int32),
        ],
        compiler_params=pltpu.CompilerParams(
            dimension_semantics=("parallel",)),
    )(r, g, act, e, e2)


_NW = 32
_BPW = _M // _NW
_CH = 96
_NCH = _BPW // _CH


def _gather_body(tab_hbm, idx_hbm, out_hbm, idx_v, rows_v, sem):
    c = lax.axis_index("c")
    s = lax.axis_index("s")
    wid = s * 2 + c
    base = wid * _BPW
    for j in range(_NCH):
        pltpu.sync_copy(idx_hbm.at[pl.ds(base + j * _CH, _CH)], idx_v.at[j])
    copies = [
        pltpu.async_copy(tab_hbm.at[idx_v.at[j]], rows_v.at[j], sem)
        for j in range(_NCH)
    ]
    for cp in copies:
        cp.wait()
    for j in range(_NCH):
        pltpu.sync_copy(rows_v.at[j], out_hbm.at[pl.ds(base + j * _CH, _CH)])


@functools.cache
def _gather_kernel():
    return pl.kernel(
        _gather_body,
        out_type=jax.ShapeDtypeStruct((_M, _D_LAT), jnp.float32),
        mesh=plsc.VectorSubcoreMesh(core_axis_name="c", subcore_axis_name="s"),
        scratch_types=[
            pltpu.VMEM((_NCH, _CH), jnp.int32),
            pltpu.VMEM((_NCH, _CH, _D_LAT), jnp.float32),
            pltpu.SemaphoreType.DMA,
        ],
    )


def _gather(tab, idx):
    return _gather_kernel()(tab, idx)


def _final_body(z0, z1, z2, r3, g3, act_ref, w_ref, b_ref, l0, l1, l2,
                zq_ref, zqout_ref, loss_ref, zql3_ref):
    act = act_ref[0] != 0
    zql3, l3, _ = _upd_core(r3[...], g3[...], act)
    zql3_ref[...] = zql3
    zo = ((z0[...] + z1[...]) + z2[...]) + zql3
    zqout_ref[...] = zo
    zq_ref[...] = (
        jnp.dot(zo, w_ref[...], preferred_element_type=jnp.float32) + b_ref[...]
    )
    loss_ref[...] = (((l0[...] + l1[...]) + l2[...]) + l3) / 4.0


def _final(zqls, r3, g3, act, w, b, lossls):
    tm = 1024
    zspec = pl.BlockSpec((tm, _D_LAT), lambda i: (i, 0))
    lspec = pl.BlockSpec((tm, 1), lambda i: (i, 0))
    return pl.pallas_call(
        _final_body,
        grid=(_M // tm,),
        in_specs=[zspec, zspec, zspec, zspec, zspec,
                  pl.BlockSpec(memory_space=pltpu.SMEM),
                  pl.BlockSpec((_D_LAT, _D_IN), lambda i: (0, 0)),
                  pl.BlockSpec((1, _D_IN), lambda i: (0, 0)),
                  lspec, lspec, lspec],
        out_specs=[
            pl.BlockSpec((tm, _D_IN), lambda i: (i, 0)),
            pl.BlockSpec((tm, _D_LAT), lambda i: (i, 0)),
            lspec,
            zspec,
        ],
        out_shape=[
            jax.ShapeDtypeStruct((_M, _D_IN), jnp.float32),
            jax.ShapeDtypeStruct((_M, _D_LAT), jnp.float32),
            jax.ShapeDtypeStruct((_M, 1), jnp.float32),
            jax.ShapeDtypeStruct((_M, _D_LAT), jnp.float32),
        ],
    )(*zqls, r3, g3, act, w, b, *lossls)


def kernel(z, W_q, b_q, codebooks, W_post, b_post, use_codebook_num=4):
    z2 = z.astype(jnp.float32).reshape(_M, _D_IN)
    e2s = _e2_all(codebooks)

    acts = [
        (jnp.asarray(h) < use_codebook_num).astype(jnp.int32).reshape(1)
        for h in range(_NUM_CB)
    ]
    r0, idx = _proj_dist0(z2, W_q, b_q.reshape(1, _D_LAT), codebooks[0], e2s[0])
    g = _gather(codebooks[0], idx)
    idx0 = jnp.where(acts[0][0] != 0, idx, jnp.zeros_like(idx))

    zqls, lossls = [], []
    r, gprev = r0, g
    for h in range(1, _NUM_CB):
        zql, lossl, r, idx = _upd_dist(r, gprev, acts[h - 1], codebooks[h], e2s[h])
        gprev = _gather(codebooks[h], idx)
        zqls.append(zql)
        lossls.append(lossl)

    zq, zqout, loss, zql3 = _final(
        zqls, r, gprev, acts[_NUM_CB - 1],
        W_post, b_post.reshape(1, _D_IN), lossls,
    )
    zqls.append(zql3)

    z_q = zq.reshape(_B, _N, _D_IN)
    zq_cat = jnp.stack([a.reshape(_B, _N, _D_LAT) for a in zqls], axis=1)
    z_q_out = zqout.reshape(_B, _N, _D_LAT)
    loss = loss.reshape(_B, _N)
    return (z_q, idx0, loss, zq_cat, z_q_out)


# speedup vs baseline: 1.2314x; 1.2314x over previous
import functools

import jax
import jax.numpy as jnp
from jax import lax
from jax.experimental import pallas as pl
from jax.experimental.pallas import tpu as pltpu
from jax.experimental.pallas import tpu_sc as plsc

_B, _N, _D_IN, _D_LAT, _K, _NUM_CB = 16, 576, 768, 256, 8192, 4
_BETA = 0.25
_M = _B * _N


def _proj_in_body(z_ref, w_ref, b_ref, o_ref):
    o_ref[...] = (
        jnp.dot(z_ref[...], w_ref[...], preferred_element_type=jnp.float32)
        + b_ref[...]
    )


def _proj_in(z2, w, b):
    tm = 1024
    return pl.pallas_call(
        _proj_in_body,
        grid=(_M // tm,),
        in_specs=[
            pl.BlockSpec((tm, _D_IN), lambda i: (i, 0)),
            pl.BlockSpec((_D_IN, _D_LAT), lambda i: (0, 0)),
            pl.BlockSpec((1, _D_LAT), lambda i: (0, 0)),
        ],
        out_specs=pl.BlockSpec((tm, _D_LAT), lambda i: (i, 0)),
        out_shape=jax.ShapeDtypeStruct((_M, _D_LAT), jnp.float32),
    )(z2, w, b)


def _e2_body(cb_ref, e2_ref):
    e = cb_ref[0]
    e2_ref[0, 0, :] = jnp.sum(e * e, axis=1)


def _e2_all(codebooks):
    out = pl.pallas_call(
        _e2_body,
        grid=(_NUM_CB,),
        in_specs=[pl.BlockSpec((1, _K, _D_LAT), lambda h: (h, 0, 0))],
        out_specs=pl.BlockSpec((1, 1, _K), lambda h: (h, 0, 0)),
        out_shape=jax.ShapeDtypeStruct((_NUM_CB, 1, _K), jnp.float32),
    )(codebooks)
    return [out[h] for h in range(_NUM_CB)]


_W = 128
_C = _K // _W


def _dist_core(rn, e_ref, e2_ref, idx_ref):
    a = jnp.sum(rn * rn, axis=1, keepdims=True)
    rs = rn * (-2.0)
    s2 = lax.dot_general(
        rs, e_ref[...], (((1,), (1,)), ((), ())),
        preferred_element_type=jnp.float32,
    )
    d = (a + e2_ref[...]) + s2
    big = float(_K)
    vals = [d[:, c * _W:(c + 1) * _W] for c in range(_C)]
    idxs = []
    nv, ni = [], []
    for i in range(_C // 2):
        va, vb = vals[2 * i], vals[2 * i + 1]
        nv.append(jnp.minimum(va, vb))
        ni.append(jnp.where(va <= vb, float(2 * i), float(2 * i + 1)))
    vals, idxs = nv, ni
    while len(vals) > 1:
        nv, ni = [], []
        for i in range(len(vals) // 2):
            va, vb = vals[2 * i], vals[2 * i + 1]
            nv.append(jnp.minimum(va, vb))
            ni.append(jnp.where(va <= vb, idxs[2 * i], idxs[2 * i + 1]))
        vals, idxs = nv, ni
    colmin, runc = vals[0], idxs[0]
    rowmin = jnp.min(colmin, axis=1, keepdims=True)
    lane = lax.broadcasted_iota(jnp.int32, (1, _W), 1).astype(jnp.float32)
    key = jnp.where(colmin == rowmin, runc * float(_W) + lane, big * big)
    idxf = jnp.min(key, axis=1)
    idx_ref[...] = idxf.astype(jnp.int32)


def _upd_core(r, g, act):
    delta = g - r
    zql = jnp.where(act, r + delta, jnp.zeros_like(r))
    m = jnp.mean(delta * delta, axis=1, keepdims=True)
    lossl = jnp.where(act, m + m * _BETA, jnp.zeros_like(m))
    return zql, lossl, r - zql


_TM = 1024


def _proj_dist0_body(z_ref, w_ref, b_ref, e_ref, e2_ref, r_ref, idx_ref):
    r0 = (
        jnp.dot(z_ref[...], w_ref[...], preferred_element_type=jnp.float32)
        + b_ref[...]
    )
    r_ref[...] = r0
    _dist_core(r0, e_ref, e2_ref, idx_ref)


def _proj_dist0(z2, w, b, e, e2):
    return pl.pallas_call(
        _proj_dist0_body,
        grid=(_M // _TM,),
        in_specs=[
            pl.BlockSpec((_TM, _D_IN), lambda i: (i, 0)),
            pl.BlockSpec((_D_IN, _D_LAT), lambda i: (0, 0)),
            pl.BlockSpec((1, _D_LAT), lambda i: (0, 0)),
            pl.BlockSpec((_K, _D_LAT), lambda i: (0, 0)),
            pl.BlockSpec((1, _K), lambda i: (0, 0)),
        ],
        out_specs=[
            pl.BlockSpec((_TM, _D_LAT), lambda i: (i, 0)),
            pl.BlockSpec((_TM,), lambda i: (i,)),
        ],
        out_shape=[
            jax.ShapeDtypeStruct((_M, _D_LAT), jnp.float32),
            jax.ShapeDtypeStruct((_M,), jnp.int32),
        ],
        compiler_params=pltpu.CompilerParams(
            dimension_semantics=("parallel",)),
    )(z2, w, b, e, e2)


def _upd_dist_body(r_ref, g_ref, act_ref, e_ref, e2_ref,
                   zql_ref, loss_ref, rn_ref, idx_ref):
    act = act_ref[0] != 0
    zql, lossl, rn = _upd_core(r_ref[...], g_ref[...], act)
    zql_ref[...] = zql
    loss_ref[...] = lossl
    rn_ref[...] = rn
    _dist_core(rn, e_ref, e2_ref, idx_ref)


def _upd_dist(r, g, act, e, e2):
    return pl.pallas_call(
        _upd_dist_body,
        grid=(_M // _TM,),
        in_specs=[
            pl.BlockSpec((_TM, _D_LAT), lambda i: (i, 0)),
            pl.BlockSpec((_TM, _D_LAT), lambda i: (i, 0)),
            pl.BlockSpec(memory_space=pltpu.SMEM),
            pl.BlockSpec((_K, _D_LAT), lambda i: (0, 0)),
            pl.BlockSpec((1, _K), lambda i: (0, 0)),
        ],
        out_specs=[
            pl.BlockSpec((_TM, _D_LAT), lambda i: (i, 0)),
            pl.BlockSpec((_TM, 1), lambda i: (i, 0)),
            pl.BlockSpec((_TM, _D_LAT), lambda i: (i, 0)),
            pl.BlockSpec((_TM,), lambda i: (i,)),
        ],
        out_shape=[
            jax.ShapeDtypeStruct((_M, _D_LAT), jnp.float32),
            jax.ShapeDtypeStruct((_M, 1), jnp.float32),
            jax.ShapeDtypeStruct((_M, _D_LAT), jnp.float32),
            jax.ShapeDtypeStruct((_M,), jnp.int32),
        ],
        compiler_params=pltpu.CompilerParams(
            dimension_semantics=("parallel",)),
    )(r, g, act, e, e2)


_NW = 32
_BPW = _M // _NW
_CH = 96
_NCH = _BPW // _CH


def _gather_body(tab_hbm, idx_hbm, out_hbm, idx_v, rows_v, sem):
    c = lax.axis_index("c")
    s = lax.axis_index("s")
    wid = s * 2 + c
    base = wid * _BPW
    for j in range(_NCH):
        pltpu.sync_copy(idx_hbm.at[pl.ds(base + j * _CH, _CH)], idx_v.at[j])
    copies = [
        pltpu.async_copy(tab_hbm.at[idx_v.at[j]], rows_v.at[j], sem)
        for j in range(_NCH)
    ]
    for cp in copies:
        cp.wait()
    for j in range(_NCH):
        pltpu.sync_copy(rows_v.at[j], out_hbm.at[pl.ds(base + j * _CH, _CH)])


@functools.cache
def _gather_kernel():
    return pl.kernel(
        _gather_body,
        out_type=jax.ShapeDtypeStruct((_M, _D_LAT), jnp.float32),
        mesh=plsc.VectorSubcoreMesh(core_axis_name="c", subcore_axis_name="s"),
        scratch_types=[
            pltpu.VMEM((_NCH, _CH), jnp.int32),
            pltpu.VMEM((_NCH, _CH, _D_LAT), jnp.float32),
            pltpu.SemaphoreType.DMA,
        ],
    )


def _gather(tab, idx):
    return _gather_kernel()(tab, idx)


def _final_body(z0, z1, z2, r3, g3, act_ref, w_ref, b_ref, l0, l1, l2,
                zq_ref, zqout_ref, loss_ref, zql3_ref):
    act = act_ref[0] != 0
    zql3, l3, _ = _upd_core(r3[...], g3[...], act)
    zql3_ref[...] = zql3
    zo = ((z0[...] + z1[...]) + z2[...]) + zql3
    zqout_ref[...] = zo
    zq_ref[...] = (
        jnp.dot(zo, w_ref[...], preferred_element_type=jnp.float32) + b_ref[...]
    )
    loss_ref[...] = (((l0[...] + l1[...]) + l2[...]) + l3) / 4.0


def _final(zqls, r3, g3, act, w, b, lossls):
    tm = 1024
    zspec = pl.BlockSpec((tm, _D_LAT), lambda i: (i, 0))
    lspec = pl.BlockSpec((tm, 1), lambda i: (i, 0))
    return pl.pallas_call(
        _final_body,
        grid=(_M // tm,),
        in_specs=[zspec, zspec, zspec, zspec, zspec,
                  pl.BlockSpec(memory_space=pltpu.SMEM),
                  pl.BlockSpec((_D_LAT, _D_IN), lambda i: (0, 0)),
                  pl.BlockSpec((1, _D_IN), lambda i: (0, 0)),
                  lspec, lspec, lspec],
        out_specs=[
            pl.BlockSpec((tm, _D_IN), lambda i: (i, 0)),
            pl.BlockSpec((tm, _D_LAT), lambda i: (i, 0)),
            lspec,
            zspec,
        ],
        out_shape=[
            jax.ShapeDtypeStruct((_M, _D_IN), jnp.float32),
            jax.ShapeDtypeStruct((_M, _D_LAT), jnp.float32),
            jax.ShapeDtypeStruct((_M, 1), jnp.float32),
            jax.ShapeDtypeStruct((_M, _D_LAT), jnp.float32),
        ],
    )(*zqls, r3, g3, act, w, b, *lossls)


def kernel(z, W_q, b_q, codebooks, W_post, b_post, use_codebook_num=4):
    z2 = z.astype(jnp.float32).reshape(_M, _D_IN)
    e2s = _e2_all(codebooks)

    acts = [
        (jnp.asarray(h) < use_codebook_num).astype(jnp.int32).reshape(1)
        for h in range(_NUM_CB)
    ]
    r0, idx = _proj_dist0(z2, W_q, b_q.reshape(1, _D_LAT), codebooks[0], e2s[0])
    g = _gather(codebooks[0], idx)
    idx0 = jnp.where(acts[0][0] != 0, idx, jnp.zeros_like(idx))

    zqls, lossls = [], []
    r, gprev = r0, g
    for h in range(1, _NUM_CB):
        zql, lossl, r, idx = _upd_dist(r, gprev, acts[h - 1], codebooks[h], e2s[h])
        gprev = _gather(codebooks[h], idx)
        zqls.append(zql)
        lossls.append(lossl)

    zq, zqout, loss, zql3 = _final(
        zqls, r, gprev, acts[_NUM_CB - 1],
        W_post, b_post.reshape(1, _D_IN), lossls,
    )
    zqls.append(zql3)

    z_q = zq.reshape(_B, _N, _D_IN)
    zq_cat = jnp.stack([a.reshape(_B, _N, _D_LAT) for a in zqls], axis=1)
    z_q_out = zqout.reshape(_B, _N, _D_LAT)
    loss = loss.reshape(_B, _N)
    return (z_q, idx0, loss, zq_cat, z_q_out)


# speedup vs baseline: 1.2332x; 1.0014x over previous
import functools

import jax
import jax.numpy as jnp
from jax import lax
from jax.experimental import pallas as pl
from jax.experimental.pallas import tpu as pltpu
from jax.experimental.pallas import tpu_sc as plsc

_B, _N, _D_IN, _D_LAT, _K, _NUM_CB = 16, 576, 768, 256, 8192, 4
_BETA = 0.25
_M = _B * _N


def _proj_in_body(z_ref, w_ref, b_ref, o_ref):
    o_ref[...] = (
        jnp.dot(z_ref[...], w_ref[...], preferred_element_type=jnp.float32)
        + b_ref[...]
    )


def _proj_in(z2, w, b):
    tm = 1024
    return pl.pallas_call(
        _proj_in_body,
        grid=(_M // tm,),
        in_specs=[
            pl.BlockSpec((tm, _D_IN), lambda i: (i, 0)),
            pl.BlockSpec((_D_IN, _D_LAT), lambda i: (0, 0)),
            pl.BlockSpec((1, _D_LAT), lambda i: (0, 0)),
        ],
        out_specs=pl.BlockSpec((tm, _D_LAT), lambda i: (i, 0)),
        out_shape=jax.ShapeDtypeStruct((_M, _D_LAT), jnp.float32),
    )(z2, w, b)


def _e2_body(cb_ref, e2_ref):
    e = cb_ref[0]
    e2_ref[0, 0, :] = jnp.sum(e * e, axis=1)


def _e2_all(codebooks):
    out = pl.pallas_call(
        _e2_body,
        grid=(_NUM_CB,),
        in_specs=[pl.BlockSpec((1, _K, _D_LAT), lambda h: (h, 0, 0))],
        out_specs=pl.BlockSpec((1, 1, _K), lambda h: (h, 0, 0)),
        out_shape=jax.ShapeDtypeStruct((_NUM_CB, 1, _K), jnp.float32),
    )(codebooks)
    return [out[h] for h in range(_NUM_CB)]


_W = 128
_C = _K // _W


def _dist_core(rn, e_ref, e2_ref, idx_ref):
    a = jnp.sum(rn * rn, axis=1, keepdims=True)
    rs = rn * (-2.0)
    s2 = lax.dot_general(
        rs, e_ref[...], (((1,), (1,)), ((), ())),
        preferred_element_type=jnp.float32,
    )
    e2r = e2_ref[...]
    big = float(_K)

    def dslice(c):
        return (a + e2r[:, c * _W:(c + 1) * _W]) + s2[:, c * _W:(c + 1) * _W]

    nv, ni = [], []
    for i in range(_C // 2):
        va, vb = dslice(2 * i), dslice(2 * i + 1)
        nv.append(jnp.minimum(va, vb))
        ni.append(jnp.where(va <= vb, float(2 * i), float(2 * i + 1)))
    vals, idxs = nv, ni
    while len(vals) > 1:
        nv, ni = [], []
        for i in range(len(vals) // 2):
            va, vb = vals[2 * i], vals[2 * i + 1]
            nv.append(jnp.minimum(va, vb))
            ni.append(jnp.where(va <= vb, idxs[2 * i], idxs[2 * i + 1]))
        vals, idxs = nv, ni
    colmin, runc = vals[0], idxs[0]
    rowmin = jnp.min(colmin, axis=1, keepdims=True)
    lane = lax.broadcasted_iota(jnp.int32, (1, _W), 1).astype(jnp.float32)
    key = jnp.where(colmin == rowmin, runc * float(_W) + lane, big * big)
    idxf = jnp.min(key, axis=1)
    idx_ref[...] = idxf.astype(jnp.int32)


def _upd_core(r, g, act):
    delta = g - r
    zql = jnp.where(act, r + delta, jnp.zeros_like(r))
    m = jnp.mean(delta * delta, axis=1, keepdims=True)
    lossl = jnp.where(act, m + m * _BETA, jnp.zeros_like(m))
    return zql, lossl, r - zql


_TM = 1024


def _proj_dist0_body(z_ref, w_ref, b_ref, e_ref, e2_ref, r_ref, idx_ref):
    r0 = (
        jnp.dot(z_ref[...], w_ref[...], preferred_element_type=jnp.float32)
        + b_ref[...]
    )
    r_ref[...] = r0
    _dist_core(r0, e_ref, e2_ref, idx_ref)


def _proj_dist0(z2, w, b, e, e2):
    return pl.pallas_call(
        _proj_dist0_body,
        grid=(_M // _TM,),
        in_specs=[
            pl.BlockSpec((_TM, _D_IN), lambda i: (i, 0)),
            pl.BlockSpec((_D_IN, _D_LAT), lambda i: (0, 0)),
            pl.BlockSpec((1, _D_LAT), lambda i: (0, 0)),
            pl.BlockSpec((_K, _D_LAT), lambda i: (0, 0)),
            pl.BlockSpec((1, _K), lambda i: (0, 0)),
        ],
        out_specs=[
            pl.BlockSpec((_TM, _D_LAT), lambda i: (i, 0)),
            pl.BlockSpec((_TM,), lambda i: (i,)),
        ],
        out_shape=[
            jax.ShapeDtypeStruct((_M, _D_LAT), jnp.float32),
            jax.ShapeDtypeStruct((_M,), jnp.int32),
        ],
        compiler_params=pltpu.CompilerParams(
            dimension_semantics=("parallel",)),
    )(z2, w, b, e, e2)


def _upd_dist_body(r_ref, g_ref, act_ref, e_ref, e2_ref,
                   zql_ref, loss_ref, rn_ref, idx_ref):
    act = act_ref[0] != 0
    zql, lossl, rn = _upd_core(r_ref[...], g_ref[...], act)
    zql_ref[...] = zql
    loss_ref[...] = lossl
    rn_ref[...] = rn
    _dist_core(rn, e_ref, e2_ref, idx_ref)


def _upd_dist(r, g, act, e, e2):
    return pl.pallas_call(
        _upd_dist_body,
        grid=(_M // _TM,),
        in_specs=[
            pl.BlockSpec((_TM, _D_LAT), lambda i: (i, 0)),
            pl.BlockSpec((_TM, _D_LAT), lambda i: (i, 0)),
            pl.BlockSpec(memory_space=pltpu.SMEM),
            pl.BlockSpec((_K, _D_LAT), lambda i: (0, 0)),
            pl.BlockSpec((1, _K), lambda i: (0, 0)),
        ],
        out_specs=[
            pl.BlockSpec((_TM, _D_LAT), lambda i: (i, 0)),
            pl.BlockSpec((_TM, 1), lambda i: (i, 0)),
            pl.BlockSpec((_TM, _D_LAT), lambda i: (i, 0)),
            pl.BlockSpec((_TM,), lambda i: (i,)),
        ],
        out_shape=[
            jax.ShapeDtypeStruct((_M, _D_LAT), jnp.float32),
            jax.ShapeDtypeStruct((_M, 1), jnp.float32),
            jax.ShapeDtypeStruct((_M, _D_LAT), jnp.float32),
            jax.ShapeDtypeStruct((_M,), jnp.int32),
        ],
        compiler_params=pltpu.CompilerParams(
            dimension_semantics=("parallel",)),
    )(r, g, act, e, e2)


_NW = 32
_BPW = _M // _NW
_CH = 96
_NCH = _BPW // _CH


def _gather_body(tab_hbm, idx_hbm, out_hbm, idx_v, rows_v, sem):
    c = lax.axis_index("c")
    s = lax.axis_index("s")
    wid = s * 2 + c
    base = wid * _BPW
    for j in range(_NCH):
        pltpu.sync_copy(idx_hbm.at[pl.ds(base + j * _CH, _CH)], idx_v.at[j])
    copies = [
        pltpu.async_copy(tab_hbm.at[idx_v.at[j]], rows_v.at[j], sem)
        for j in range(_NCH)
    ]
    for cp in copies:
        cp.wait()
    for j in range(_NCH):
        pltpu.sync_copy(rows_v.at[j], out_hbm.at[pl.ds(base + j * _CH, _CH)])


@functools.cache
def _gather_kernel():
    return pl.kernel(
        _gather_body,
        out_type=jax.ShapeDtypeStruct((_M, _D_LAT), jnp.float32),
        mesh=plsc.VectorSubcoreMesh(core_axis_name="c", subcore_axis_name="s"),
        scratch_types=[
            pltpu.VMEM((_NCH, _CH), jnp.int32),
            pltpu.VMEM((_NCH, _CH, _D_LAT), jnp.float32),
            pltpu.SemaphoreType.DMA,
        ],
    )


def _gather(tab, idx):
    return _gather_kernel()(tab, idx)


def _final_body(z0, z1, z2, r3, g3, act_ref, w_ref, b_ref, l0, l1, l2,
                zq_ref, zqout_ref, loss_ref, zql3_ref):
    act = act_ref[0] != 0
    zql3, l3, _ = _upd_core(r3[...], g3[...], act)
    zql3_ref[...] = zql3
    zo = ((z0[...] + z1[...]) + z2[...]) + zql3
    zqout_ref[...] = zo
    zq_ref[...] = (
        jnp.dot(zo, w_ref[...], preferred_element_type=jnp.float32) + b_ref[...]
    )
    loss_ref[...] = (((l0[...] + l1[...]) + l2[...]) + l3) / 4.0


def _final(zqls, r3, g3, act, w, b, lossls):
    tm = 1024
    zspec = pl.BlockSpec((tm, _D_LAT), lambda i: (i, 0))
    lspec = pl.BlockSpec((tm, 1), lambda i: (i, 0))
    return pl.pallas_call(
        _final_body,
        grid=(_M // tm,),
        in_specs=[zspec, zspec, zspec, zspec, zspec,
                  pl.BlockSpec(memory_space=pltpu.SMEM),
                  pl.BlockSpec((_D_LAT, _D_IN), lambda i: (0, 0)),
                  pl.BlockSpec((1, _D_IN), lambda i: (0, 0)),
                  lspec, lspec, lspec],
        out_specs=[
            pl.BlockSpec((tm, _D_IN), lambda i: (i, 0)),
            pl.BlockSpec((tm, _D_LAT), lambda i: (i, 0)),
            lspec,
            zspec,
        ],
        out_shape=[
            jax.ShapeDtypeStruct((_M, _D_IN), jnp.float32),
            jax.ShapeDtypeStruct((_M, _D_LAT), jnp.float32),
            jax.ShapeDtypeStruct((_M, 1), jnp.float32),
            jax.ShapeDtypeStruct((_M, _D_LAT), jnp.float32),
        ],
    )(*zqls, r3, g3, act, w, b, *lossls)


def kernel(z, W_q, b_q, codebooks, W_post, b_post, use_codebook_num=4):
    z2 = z.astype(jnp.float32).reshape(_M, _D_IN)
    e2s = _e2_all(codebooks)

    acts = [
        (jnp.asarray(h) < use_codebook_num).astype(jnp.int32).reshape(1)
        for h in range(_NUM_CB)
    ]
    r0, idx = _proj_dist0(z2, W_q, b_q.reshape(1, _D_LAT), codebooks[0], e2s[0])
    g = _gather(codebooks[0], idx)
    idx0 = jnp.where(acts[0][0] != 0, idx, jnp.zeros_like(idx))

    zqls, lossls = [], []
    r, gprev = r0, g
    for h in range(1, _NUM_CB):
        zql, lossl, r, idx = _upd_dist(r, gprev, acts[h - 1], codebooks[h], e2s[h])
        gprev = _gather(codebooks[h], idx)
        zqls.append(zql)
        lossls.append(lossl)

    zq, zqout, loss, zql3 = _final(
        zqls, r, gprev, acts[_NUM_CB - 1],
        W_post, b_post.reshape(1, _D_IN), lossls,
    )
    zqls.append(zql3)

    z_q = zq.reshape(_B, _N, _D_IN)
    zq_cat = jnp.stack([a.reshape(_B, _N, _D_LAT) for a in zqls], axis=1)
    z_q_out = zqout.reshape(_B, _N, _D_LAT)
    loss = loss.reshape(_B, _N)
    return (z_q, idx0, loss, zq_cat, z_q_out)
